# baseline (speedup 1.0000x reference)
import functools

import jax
import jax.numpy as jnp
from jax import lax
from jax.experimental import pallas as pl
from jax.experimental.pallas import tpu as pltpu
from jax.experimental.pallas import tpu_sc as plsc

_B = 16384
_EMB = 64
_FEAT = 10
_NC = 2
_NS = 16
_NW = _NC * _NS
_BPW = _B // _NW
_K = 16


_NUSERS = 1000000
_W = 256
_WSH = 8
_NWIN_FULL = _NUSERS // _W
_TAILW = _NUSERS - _NWIN_FULL * _W
_CAP_CH = 32
_CAP_ST = 768
_SENT = jnp.int32(1 << 30)


def _make_sc_gather():
    mesh = plsc.VectorSubcoreMesh(core_axis_name="c", subcore_axis_name="s")
    i32 = jnp.int32

    @functools.partial(
        pl.kernel,
        mesh=mesh,
        compiler_params=pltpu.CompilerParams(needs_layout_passes=False),
        out_type=[
            jax.ShapeDtypeStruct((_B * _EMB,), jnp.float32),
            jax.ShapeDtypeStruct((_B, _EMB), jnp.float32),
        ],
        scratch_types=[
            pltpu.VMEM((_B,), i32),
            pltpu.VMEM((_CAP_ST,), i32),
            pltpu.VMEM((_CAP_ST,), i32),
            pltpu.VMEM((_CAP_CH,), i32),
            pltpu.VMEM((_CAP_CH,), i32),
            pltpu.VMEM((_EMB, 2 * _W), jnp.float32),
            pltpu.VMEM((_EMB, _TAILW), jnp.float32),
            pltpu.VMEM((_CAP_ST * _EMB,), jnp.float32),
            pltpu.VMEM((_CAP_ST,), i32),
            pltpu.VMEM((_BPW,), i32),
            pltpu.VMEM((_K, _EMB), jnp.float32),
            pltpu.SemaphoreType.DMA,
            pltpu.SemaphoreType.DMA,
            pltpu.SemaphoreType.DMA,
            pltpu.SemaphoreType.DMA,
        ],
    )
    def gather2(uidx_hbm, utabT_hbm, utail_hbm, gidx_hbm, gtab_hbm,
                uout_hbm, gout_hbm,
                uidx_v, ridx_v, rpos_v, cidx_v, cpos_v, win_v, tail_v,
                stg_v, spos_v, gidx_v, grow_v, sem0, sem1, gsem, osem):
        wid = lax.axis_index("s") * _NC + lax.axis_index("c")
        lanes = lax.iota(i32, 16)
        pltpu.sync_copy(uidx_hbm, uidx_v)

        def snt(g, carry):
            ridx_v[pl.ds(g * 16, 16)] = jnp.full((16,), _SENT, i32)
            return carry
        lax.fori_loop(0, _CAP_ST // 16, snt, 0)

        def filt(g, cnt):
            v = uidx_v[pl.ds(g * 16, 16)]
            m = jnp.bitwise_and(lax.shift_right_logical(v, _WSH), 31) == wid
            cs = plsc.cumsum(m.astype(i32))
            posn = jnp.minimum(cnt + cs - 1, _CAP_ST - 1)
            plsc.store_scatter(ridx_v, [posn], v, mask=m)
            plsc.store_scatter(rpos_v, [posn], g * 16 + lanes, mask=m)
            return jnp.minimum(cnt + cs[15], _CAP_ST - 1)
        lax.fori_loop(0, _B // 16, filt, jnp.int32(0))

        def fire(j, slot):
            cb = pl.multiple_of((wid + 32 * j) * _W, _W)
            pltpu.make_async_copy(
                utabT_hbm.at[:, pl.ds(cb, _W)],
                win_v.at[:, pl.ds(slot * _W, _W)],
                sem0 if slot == 0 else sem1).start()

        def wwait(slot):
            pltpu.make_async_copy(
                utabT_hbm.at[:, pl.ds(0, _W)],
                win_v.at[:, pl.ds(slot * _W, _W)],
                sem0 if slot == 0 else sem1).wait()

        def extract(k, base, H, buf):
            def resc(g, n):
                rv = ridx_v[pl.ds(g * 16, 16)]
                mc = lax.shift_right_logical(rv, _WSH) == k
                cs = plsc.cumsum(mc.astype(i32))
                posn = jnp.minimum(n + cs - 1, _CAP_CH - 1)
                plsc.store_scatter(cidx_v, [posn], rv, mask=mc)
                pv = rpos_v[pl.ds(g * 16, 16)]
                plsc.store_scatter(cpos_v, [posn], pv, mask=mc)
                return jnp.minimum(n + cs[15], _CAP_CH - 1)
            n = lax.fori_loop(0, _CAP_ST // 16, resc, jnp.int32(0))

            def hit(i, H):
                uidx = plsc.load_gather(cidx_v, [jnp.full((16,), i, i32)])[0]
                upos = plsc.load_gather(cpos_v, [jnp.full((16,), i, i32)])[0]
                u = jnp.full((16,), uidx - k * _W + base, i32)
                Hc = jnp.minimum(H, _CAP_ST - 1)
                for q in range(4):
                    vals = plsc.load_gather(buf, [lanes + q * 16, u])
                    off = pl.multiple_of(Hc * _EMB, _EMB) + q * 16
                    stg_v[pl.ds(off, 16)] = vals
                plsc.store_scatter(spos_v, [jnp.full((16,), Hc, i32)],
                                   jnp.full((16,), upos, i32),
                                   mask=lanes == 0)
                return H + 1
            return lax.fori_loop(0, n, hit, H)

        ntrip = jnp.where(wid < _NWIN_FULL % 32,
                          _NWIN_FULL // 32 + 1, _NWIN_FULL // 32)
        fire(jnp.int32(0), 0)

        def body(c, H):
            nxt = c + 1
            even = (c % 2) == 0

            @pl.when((nxt < ntrip) & even)
            def _():
                fire(nxt, 1)

            @pl.when((nxt < ntrip) & jnp.logical_not(even))
            def _():
                fire(nxt, 0)

            k = wid + 32 * c

            def p0():
                wwait(0)
                return extract(k, 0, H, win_v)

            def p1():
                wwait(1)
                return extract(k, _W, H, win_v)
            return lax.cond(even, p0, p1)

        H = lax.fori_loop(0, ntrip, body, jnp.int32(0))

        def ptail():
            pltpu.make_async_copy(utail_hbm, tail_v, sem0).start()
            pltpu.make_async_copy(utail_hbm, tail_v, sem0).wait()
            return extract(jnp.int32(_NWIN_FULL), 0, H, tail_v)
        H = lax.cond(wid == (_NWIN_FULL & 31), ptail, lambda: H)

        Hpad = jnp.minimum((H + 15) & ~15, _CAP_ST)
        pos0 = spos_v[pl.ds(0, 16)][0]

        def pad(i, carry):
            for q in range(4):
                stg_v[pl.ds(pl.multiple_of(i * _EMB, _EMB) + q * 16, 16)] = (
                    stg_v[pl.ds(q * 16, 16)])
            plsc.store_scatter(spos_v, [jnp.full((16,), i, i32)],
                               jnp.full((16,), pos0, i32),
                               mask=lanes == 0)
            return carry
        lax.fori_loop(H, Hpad, pad, 0)

        def drain(b, carry):
            o = b * 16
            pv = spos_v[pl.ds(pl.multiple_of(o, 16), 16)] * _EMB
            for j in range(16):
                pltpu.make_async_copy(
                    stg_v.at[pl.ds(pl.multiple_of((o + j) * _EMB, _EMB), _EMB)],
                    uout_hbm.at[pl.ds(pl.multiple_of(pv[j], _EMB), _EMB)],
                    osem).start()
            for j in range(16):
                pltpu.make_async_copy(
                    stg_v.at[pl.ds(0, _EMB)],
                    uout_hbm.at[pl.ds(0, _EMB)], osem).wait()
            return carry
        lax.fori_loop(0, Hpad // 16, drain, 0)

        gbase = wid * _BPW
        pltpu.sync_copy(gidx_hbm.at[pl.ds(gbase, _BPW)], gidx_v)

        def gchunk(c, carry):
            o = c * _K
            gv = gidx_v[pl.ds(o, _K)]
            for j in range(_K):
                pltpu.make_async_copy(
                    gtab_hbm.at[pl.ds(gv[j], 1)],
                    grow_v.at[pl.ds(j, 1)], gsem).start()
            for j in range(_K):
                pltpu.make_async_copy(
                    gtab_hbm.at[pl.ds(0, 1)],
                    grow_v.at[pl.ds(j, 1)], gsem).wait()
            pltpu.sync_copy(grow_v, gout_hbm.at[pl.ds(gbase + o, _K)])
            return carry
        lax.fori_loop(0, _BPW // _K, gchunk, 0)

    return gather2


_sc_gather_cache = []


def _sc_gather(uidx, utabT, utail, gidx, gtab):
    if not _sc_gather_cache:
        _sc_gather_cache.append(_make_sc_gather())
    return _sc_gather_cache[0](uidx, utabT, utail, gidx, gtab)


def _dotT(lhsT, rhs):
    return lax.dot_general(lhsT, rhs, (((0,), (0,)), ((), ())),
                           preferred_element_type=jnp.float32)


def _mlp_body(gfT_ref, glT_ref, umf_ref, gmf_ref,
              gw1_ref, gb1_ref, gw2_ref, gb2_ref,
              uw1_ref, ub1_ref, uw2_ref, ub2_ref,
              dw1a_ref, dw1b_ref, db1_ref, dw2_ref, db2_ref,
              out_ref):
    g1 = jnp.maximum(_dotT(gfT_ref[...], gw1_ref[...]) + gb1_ref[...], 0.0)
    genc = jnp.maximum(
        jnp.dot(g1, gw2_ref[...], preferred_element_type=jnp.float32)
        + gb2_ref[...], 0.0)
    u1 = jnp.maximum(_dotT(glT_ref[...], uw1_ref[...]) + ub1_ref[...], 0.0)
    uenc = jnp.maximum(
        jnp.dot(u1, uw2_ref[...], preferred_element_type=jnp.float32)
        + ub2_ref[...], 0.0)
    fu = umf_ref[...] + uenc
    fg = gmf_ref[...] + genc
    h = jnp.maximum(
        jnp.dot(fu, dw1a_ref[...], preferred_element_type=jnp.float32)
        + jnp.dot(fg, dw1b_ref[...], preferred_element_type=jnp.float32)
        + db1_ref[...], 0.0)
    out_ref[...] = (jnp.dot(h, dw2_ref[...], preferred_element_type=jnp.float32)
                    + db2_ref[...])


_R = 2048


def _dense(gfT, glT, umf, gmf, gW1, gb1, gW2, gb2, uW1, ub1, uW2, ub2,
           dW1a, dW1b, db1, dW2, db2):
    nblk = _B // _R

    def cols(i):
        return (0, i)

    def rows(i):
        return (i, 0)

    def whole(i):
        return (0, 0)

    col_spec_feat = pl.BlockSpec((_FEAT, _R), cols)
    row_spec_emb = pl.BlockSpec((_R, _EMB), rows)

    def wspec(a):
        return pl.BlockSpec(a.shape, whole)

    out = pl.pallas_call(
        _mlp_body,
        grid=(nblk,),
        in_specs=[
            col_spec_feat, col_spec_feat, row_spec_emb, row_spec_emb,
            wspec(gW1), wspec(gb1), wspec(gW2), wspec(gb2),
            wspec(uW1), wspec(ub1), wspec(uW2), wspec(ub2),
            wspec(dW1a), wspec(dW1b), wspec(db1), wspec(dW2), wspec(db2),
        ],
        out_specs=pl.BlockSpec((_R, 1), lambda i: (i, 0)),
        out_shape=jax.ShapeDtypeStruct((_B, 1), jnp.float32),
    )(gfT, glT, umf, gmf, gW1, gb1, gW2, gb2, uW1, ub1, uW2, ub2,
      dW1a, dW1b, db1, dW2, db2)
    return out[:, 0]


def kernel(user_input, game_input, game_features, global_features,
           user_table, game_table,
           gW1, gb1, gW2, gb2,
           uW1, ub1, uW2, ub2,
           dW1, db1, dW2, db2):
    utabT = user_table.T
    umf_flat, gmf = _sc_gather(
        user_input, utabT, utabT[:, _NWIN_FULL * _W:], game_input, game_table)
    umf = umf_flat.reshape(_B, _EMB)
    dW1a = dW1[:_EMB]
    dW1b = dW1[_EMB:]
    return _dense(
        game_features.T, global_features.T, umf, gmf,
        gW1, gb1.reshape(1, -1), gW2, gb2.reshape(1, -1),
        uW1, ub1.reshape(1, -1), uW2, ub2.reshape(1, -1),
        dW1a, dW1b, db1.reshape(1, -1), dW2, db2.reshape(1, -1))


# speedup vs baseline: 1.0926x; 1.0926x over previous
import functools

import jax
import jax.numpy as jnp
from jax import lax
from jax.experimental import pallas as pl
from jax.experimental.pallas import tpu as pltpu
from jax.experimental.pallas import tpu_sc as plsc

_B = 16384
_EMB = 64
_FEAT = 10
_NC = 2
_NS = 16
_NW = _NC * _NS
_BPW = _B // _NW
_K = 16


_NUSERS = 1000000
_W = 256
_WSH = 8
_NWIN_FULL = _NUSERS // _W
_TAILW = _NUSERS - _NWIN_FULL * _W
_CAP_CH = 32
_CAP_ST = 768
_NT = 124
_SENT = jnp.int32(1 << 30)


def _make_sc_gather():
    mesh = plsc.VectorSubcoreMesh(core_axis_name="c", subcore_axis_name="s")
    i32 = jnp.int32

    @functools.partial(
        pl.kernel,
        mesh=mesh,
        compiler_params=pltpu.CompilerParams(needs_layout_passes=False),
        out_type=[
            jax.ShapeDtypeStruct((_B * _EMB,), jnp.float32),
            jax.ShapeDtypeStruct((_B, _EMB), jnp.float32),
        ],
        scratch_types=[
            pltpu.VMEM((_B,), i32),
            pltpu.VMEM((_CAP_ST,), i32),
            pltpu.VMEM((_CAP_ST,), i32),
            pltpu.VMEM((_NT * _CAP_CH,), i32),
            pltpu.SMEM((_NT + 4,), i32),
            pltpu.VMEM((_EMB, 2 * _W), jnp.float32),
            pltpu.VMEM((_EMB, _TAILW), jnp.float32),
            pltpu.VMEM((_CAP_ST * _EMB,), jnp.float32),
            pltpu.VMEM((_CAP_ST,), i32),
            pltpu.VMEM((_BPW,), i32),
            pltpu.VMEM((_K, _EMB), jnp.float32),
            pltpu.SemaphoreType.DMA,
            pltpu.SemaphoreType.DMA,
            pltpu.SemaphoreType.DMA,
            pltpu.SemaphoreType.DMA,
        ],
    )
    def gather2(uidx_hbm, utabT_hbm, utail_hbm, gidx_hbm, gtab_hbm,
                uout_hbm, gout_hbm,
                uidx_v, ridx_v, rpos_v, bkt_v, bcnt_s, win_v, tail_v,
                stg_v, spos_v, gidx_v, grow_v, sem0, sem1, gsem, osem):
        wid = lax.axis_index("s") * _NC + lax.axis_index("c")
        lanes = lax.iota(i32, 16)
        pltpu.sync_copy(uidx_hbm, uidx_v)

        def filt(g, cnt):
            v = uidx_v[pl.ds(g * 16, 16)]
            m = jnp.bitwise_and(lax.shift_right_logical(v, _WSH), 31) == wid
            cs = plsc.cumsum(m.astype(i32))
            posn = jnp.minimum(cnt + cs - 1, _CAP_ST - 1)
            plsc.store_scatter(ridx_v, [posn], v, mask=m)
            plsc.store_scatter(rpos_v, [posn], g * 16 + lanes, mask=m)
            return jnp.minimum(cnt + cs[15], _CAP_ST - 1)
        H1 = lax.fori_loop(0, _B // 16, filt, jnp.int32(0))

        def zcnt(c, carry):
            bcnt_s[c] = 0
            return carry
        lax.fori_loop(0, _NT, zcnt, 0)

        def bins(i, carry):
            vidx = plsc.load_gather(ridx_v, [jnp.full((16,), i, i32)])[0]
            vpos = plsc.load_gather(rpos_v, [jnp.full((16,), i, i32)])[0]
            c = lax.shift_right_logical(
                lax.shift_right_logical(vidx, _WSH) - wid, 5)
            u = jnp.bitwise_and(vidx, _W - 1)
            e = jnp.bitwise_or(lax.shift_left(u, 14), vpos)
            cnt = jnp.minimum(bcnt_s[c], _CAP_CH - 1)
            plsc.store_scatter(bkt_v, [jnp.full((16,), c * _CAP_CH + cnt, i32)],
                               jnp.full((16,), e, i32), mask=lanes == 0)
            bcnt_s[c] = cnt + 1
            return carry
        lax.fori_loop(0, H1, bins, 0)

        def fire(j, slot):
            cb = pl.multiple_of((wid + 32 * j) * _W, _W)
            pltpu.make_async_copy(
                utabT_hbm.at[:, pl.ds(cb, _W)],
                win_v.at[:, pl.ds(slot * _W, _W)],
                sem0 if slot == 0 else sem1).start()

        def wwait(slot):
            pltpu.make_async_copy(
                utabT_hbm.at[:, pl.ds(0, _W)],
                win_v.at[:, pl.ds(slot * _W, _W)],
                sem0 if slot == 0 else sem1).wait()

        def extract(c, base, H, buf):
            n = bcnt_s[c]

            def hit(i, H):
                e = plsc.load_gather(
                    bkt_v, [jnp.full((16,), c * _CAP_CH + i, i32)])[0]
                upos = jnp.bitwise_and(e, (1 << 14) - 1)
                u = jnp.full((16,), lax.shift_right_logical(e, 14) + base, i32)
                Hc = jnp.minimum(H, _CAP_ST - 1)
                for q in range(4):
                    vals = plsc.load_gather(buf, [lanes + q * 16, u])
                    off = pl.multiple_of(Hc * _EMB, _EMB) + q * 16
                    stg_v[pl.ds(off, 16)] = vals
                plsc.store_scatter(spos_v, [jnp.full((16,), Hc, i32)],
                                   jnp.full((16,), upos, i32),
                                   mask=lanes == 0)
                return H + 1
            return lax.fori_loop(0, n, hit, H)

        ntrip = jnp.where(wid < _NWIN_FULL % 32,
                          _NWIN_FULL // 32 + 1, _NWIN_FULL // 32)
        fire(jnp.int32(0), 0)

        def body(c, H):
            nxt = c + 1
            even = (c % 2) == 0

            @pl.when((nxt < ntrip) & even)
            def _():
                fire(nxt, 1)

            @pl.when((nxt < ntrip) & jnp.logical_not(even))
            def _():
                fire(nxt, 0)

            def p0():
                wwait(0)
                return extract(c, 0, H, win_v)

            def p1():
                wwait(1)
                return extract(c, _W, H, win_v)
            return lax.cond(even, p0, p1)

        H = lax.fori_loop(0, ntrip, body, jnp.int32(0))

        def ptail():
            pltpu.make_async_copy(utail_hbm, tail_v, sem0).start()
            pltpu.make_async_copy(utail_hbm, tail_v, sem0).wait()
            return extract(jnp.int32((_NWIN_FULL - (_NWIN_FULL & 31)) >> 5),
                           0, H, tail_v)
        H = lax.cond(wid == (_NWIN_FULL & 31), ptail, lambda: H)

        Hpad = jnp.minimum((H + 15) & ~15, _CAP_ST)
        pos0 = spos_v[pl.ds(0, 16)][0]

        def pad(i, carry):
            for q in range(4):
                stg_v[pl.ds(pl.multiple_of(i * _EMB, _EMB) + q * 16, 16)] = (
                    stg_v[pl.ds(q * 16, 16)])
            plsc.store_scatter(spos_v, [jnp.full((16,), i, i32)],
                               jnp.full((16,), pos0, i32),
                               mask=lanes == 0)
            return carry
        lax.fori_loop(H, Hpad, pad, 0)

        def drain(b, carry):
            o = b * 16
            pv = spos_v[pl.ds(pl.multiple_of(o, 16), 16)] * _EMB
            for j in range(16):
                pltpu.make_async_copy(
                    stg_v.at[pl.ds(pl.multiple_of((o + j) * _EMB, _EMB), _EMB)],
                    uout_hbm.at[pl.ds(pl.multiple_of(pv[j], _EMB), _EMB)],
                    osem).start()
            for j in range(16):
                pltpu.make_async_copy(
                    stg_v.at[pl.ds(0, _EMB)],
                    uout_hbm.at[pl.ds(0, _EMB)], osem).wait()
            return carry
        lax.fori_loop(0, Hpad // 16, drain, 0)

        gbase = wid * _BPW
        pltpu.sync_copy(gidx_hbm.at[pl.ds(gbase, _BPW)], gidx_v)

        def gchunk(c, carry):
            o = c * _K
            gv = gidx_v[pl.ds(o, _K)]
            for j in range(_K):
                pltpu.make_async_copy(
                    gtab_hbm.at[pl.ds(gv[j], 1)],
                    grow_v.at[pl.ds(j, 1)], gsem).start()
            for j in range(_K):
                pltpu.make_async_copy(
                    gtab_hbm.at[pl.ds(0, 1)],
                    grow_v.at[pl.ds(j, 1)], gsem).wait()
            pltpu.sync_copy(grow_v, gout_hbm.at[pl.ds(gbase + o, _K)])
            return carry
        lax.fori_loop(0, _BPW // _K, gchunk, 0)

    return gather2


_sc_gather_cache = []


def _sc_gather(uidx, utabT, utail, gidx, gtab):
    if not _sc_gather_cache:
        _sc_gather_cache.append(_make_sc_gather())
    return _sc_gather_cache[0](uidx, utabT, utail, gidx, gtab)


def _dotT(lhsT, rhs):
    return lax.dot_general(lhsT, rhs, (((0,), (0,)), ((), ())),
                           preferred_element_type=jnp.float32)


def _mlp_body(gfT_ref, glT_ref, umf_ref, gmf_ref,
              gw1_ref, gb1_ref, gw2_ref, gb2_ref,
              uw1_ref, ub1_ref, uw2_ref, ub2_ref,
              dw1a_ref, dw1b_ref, db1_ref, dw2_ref, db2_ref,
              out_ref):
    g1 = jnp.maximum(_dotT(gfT_ref[...], gw1_ref[...]) + gb1_ref[...], 0.0)
    genc = jnp.maximum(
        jnp.dot(g1, gw2_ref[...], preferred_element_type=jnp.float32)
        + gb2_ref[...], 0.0)
    u1 = jnp.maximum(_dotT(glT_ref[...], uw1_ref[...]) + ub1_ref[...], 0.0)
    uenc = jnp.maximum(
        jnp.dot(u1, uw2_ref[...], preferred_element_type=jnp.float32)
        + ub2_ref[...], 0.0)
    fu = umf_ref[...] + uenc
    fg = gmf_ref[...] + genc
    h = jnp.maximum(
        jnp.dot(fu, dw1a_ref[...], preferred_element_type=jnp.float32)
        + jnp.dot(fg, dw1b_ref[...], preferred_element_type=jnp.float32)
        + db1_ref[...], 0.0)
    out_ref[...] = (jnp.dot(h, dw2_ref[...], preferred_element_type=jnp.float32)
                    + db2_ref[...])


_R = 2048


def _dense(gfT, glT, umf, gmf, gW1, gb1, gW2, gb2, uW1, ub1, uW2, ub2,
           dW1a, dW1b, db1, dW2, db2):
    nblk = _B // _R

    def cols(i):
        return (0, i)

    def rows(i):
        return (i, 0)

    def whole(i):
        return (0, 0)

    col_spec_feat = pl.BlockSpec((_FEAT, _R), cols)
    row_spec_emb = pl.BlockSpec((_R, _EMB), rows)

    def wspec(a):
        return pl.BlockSpec(a.shape, whole)

    out = pl.pallas_call(
        _mlp_body,
        grid=(nblk,),
        in_specs=[
            col_spec_feat, col_spec_feat, row_spec_emb, row_spec_emb,
            wspec(gW1), wspec(gb1), wspec(gW2), wspec(gb2),
            wspec(uW1), wspec(ub1), wspec(uW2), wspec(ub2),
            wspec(dW1a), wspec(dW1b), wspec(db1), wspec(dW2), wspec(db2),
        ],
        out_specs=pl.BlockSpec((_R, 1), lambda i: (i, 0)),
        out_shape=jax.ShapeDtypeStruct((_B, 1), jnp.float32),
    )(gfT, glT, umf, gmf, gW1, gb1, gW2, gb2, uW1, ub1, uW2, ub2,
      dW1a, dW1b, db1, dW2, db2)
    return out[:, 0]


def kernel(user_input, game_input, game_features, global_features,
           user_table, game_table,
           gW1, gb1, gW2, gb2,
           uW1, ub1, uW2, ub2,
           dW1, db1, dW2, db2):
    utabT = user_table.T
    umf_flat, gmf = _sc_gather(
        user_input, utabT, utabT[:, _NWIN_FULL * _W:], game_input, game_table)
    umf = umf_flat.reshape(_B, _EMB)
    dW1a = dW1[:_EMB]
    dW1b = dW1[_EMB:]
    return _dense(
        game_features.T, global_features.T, umf, gmf,
        gW1, gb1.reshape(1, -1), gW2, gb2.reshape(1, -1),
        uW1, ub1.reshape(1, -1), uW2, ub2.reshape(1, -1),
        dW1a, dW1b, db1.reshape(1, -1), dW2, db2.reshape(1, -1))


# speedup vs baseline: 1.2361x; 1.1313x over previous
import functools

import jax
import jax.numpy as jnp
from jax import lax
from jax.experimental import pallas as pl
from jax.experimental.pallas import tpu as pltpu
from jax.experimental.pallas import tpu_sc as plsc

_B = 16384
_EMB = 64
_FEAT = 10
_NC = 2
_NS = 16
_NW = _NC * _NS
_BPW = _B // _NW
_K = 16


_NUSERS = 1000000
_W = 256
_WSH = 8
_NWIN_FULL = _NUSERS // _W
_TAILW = _NUSERS - _NWIN_FULL * _W
_CAP_CH = 32
_CAP_ST = 768
_NT = 124
_SENT = jnp.int32(1 << 30)


def _make_sc_gather():
    mesh = plsc.VectorSubcoreMesh(core_axis_name="c", subcore_axis_name="s")
    i32 = jnp.int32

    @functools.partial(
        pl.kernel,
        mesh=mesh,
        compiler_params=pltpu.CompilerParams(needs_layout_passes=False),
        out_type=jax.ShapeDtypeStruct((_B * _EMB,), jnp.float32),
        scratch_types=[
            pltpu.VMEM((_B,), i32),
            pltpu.VMEM((_CAP_ST,), i32),
            pltpu.VMEM((_CAP_ST,), i32),
            pltpu.VMEM((_NT * _CAP_CH,), i32),
            pltpu.SMEM((_NT + 4,), i32),
            pltpu.VMEM((_EMB, 2 * _W), jnp.float32),
            pltpu.VMEM((_EMB, _TAILW), jnp.float32),
            pltpu.VMEM((_CAP_ST * _EMB,), jnp.float32),
            pltpu.VMEM((_CAP_ST,), i32),
            pltpu.SemaphoreType.DMA,
            pltpu.SemaphoreType.DMA,
            pltpu.SemaphoreType.DMA,
        ],
    )
    def gather_user(uidx_hbm, utabT_hbm, utail_hbm, uout_hbm,
                    uidx_v, ridx_v, rpos_v, bkt_v, bcnt_s, win_v, tail_v,
                    stg_v, spos_v, sem0, sem1, osem):
        wid = lax.axis_index("s") * _NC + lax.axis_index("c")
        lanes = lax.iota(i32, 16)
        pltpu.sync_copy(uidx_hbm, uidx_v)

        def filt(g, cnt):
            v = uidx_v[pl.ds(g * 16, 16)]
            m = jnp.bitwise_and(lax.shift_right_logical(v, _WSH), 31) == wid
            cs = plsc.cumsum(m.astype(i32))
            posn = jnp.minimum(cnt + cs - 1, _CAP_ST - 1)
            plsc.store_scatter(ridx_v, [posn], v, mask=m)
            plsc.store_scatter(rpos_v, [posn], g * 16 + lanes, mask=m)
            return jnp.minimum(cnt + cs[15], _CAP_ST - 1)
        H1 = lax.fori_loop(0, _B // 16, filt, jnp.int32(0))

        def zcnt(c, carry):
            bcnt_s[c] = 0
            return carry
        lax.fori_loop(0, _NT, zcnt, 0)

        def bins(i, carry):
            vidx = plsc.load_gather(ridx_v, [jnp.full((16,), i, i32)])[0]
            vpos = plsc.load_gather(rpos_v, [jnp.full((16,), i, i32)])[0]
            c = lax.shift_right_logical(
                lax.shift_right_logical(vidx, _WSH) - wid, 5)
            u = jnp.bitwise_and(vidx, _W - 1)
            e = jnp.bitwise_or(lax.shift_left(u, 14), vpos)
            cnt = jnp.minimum(bcnt_s[c], _CAP_CH - 1)
            plsc.store_scatter(bkt_v, [jnp.full((16,), c * _CAP_CH + cnt, i32)],
                               jnp.full((16,), e, i32), mask=lanes == 0)
            bcnt_s[c] = cnt + 1
            return carry
        lax.fori_loop(0, H1, bins, 0)

        def fire(j, slot):
            cb = pl.multiple_of((wid + 32 * j) * _W, _W)
            pltpu.make_async_copy(
                utabT_hbm.at[:, pl.ds(cb, _W)],
                win_v.at[:, pl.ds(slot * _W, _W)],
                sem0 if slot == 0 else sem1).start()

        def wwait(slot):
            pltpu.make_async_copy(
                utabT_hbm.at[:, pl.ds(0, _W)],
                win_v.at[:, pl.ds(slot * _W, _W)],
                sem0 if slot == 0 else sem1).wait()

        def extract(c, base, H, buf):
            n = bcnt_s[c]

            def hit(i, H):
                e = plsc.load_gather(
                    bkt_v, [jnp.full((16,), c * _CAP_CH + i, i32)])[0]
                upos = jnp.bitwise_and(e, (1 << 14) - 1)
                u = jnp.full((16,), lax.shift_right_logical(e, 14) + base, i32)
                Hc = jnp.minimum(H, _CAP_ST - 1)
                for q in range(4):
                    vals = plsc.load_gather(buf, [lanes + q * 16, u])
                    off = pl.multiple_of(Hc * _EMB, _EMB) + q * 16
                    stg_v[pl.ds(off, 16)] = vals
                plsc.store_scatter(spos_v, [jnp.full((16,), Hc, i32)],
                                   jnp.full((16,), upos, i32),
                                   mask=lanes == 0)
                return H + 1
            return lax.fori_loop(0, n, hit, H)

        ntrip = jnp.where(wid < _NWIN_FULL % 32,
                          _NWIN_FULL // 32 + 1, _NWIN_FULL // 32)
        fire(jnp.int32(0), 0)

        def body(c, H):
            nxt = c + 1
            even = (c % 2) == 0

            @pl.when((nxt < ntrip) & even)
            def _():
                fire(nxt, 1)

            @pl.when((nxt < ntrip) & jnp.logical_not(even))
            def _():
                fire(nxt, 0)

            def p0():
                wwait(0)
                return extract(c, 0, H, win_v)

            def p1():
                wwait(1)
                return extract(c, _W, H, win_v)
            return lax.cond(even, p0, p1)

        H = lax.fori_loop(0, ntrip, body, jnp.int32(0))

        def ptail():
            pltpu.make_async_copy(utail_hbm, tail_v, sem0).start()
            pltpu.make_async_copy(utail_hbm, tail_v, sem0).wait()
            return extract(jnp.int32((_NWIN_FULL - (_NWIN_FULL & 31)) >> 5),
                           0, H, tail_v)
        H = lax.cond(wid == (_NWIN_FULL & 31), ptail, lambda: H)

        Hpad = jnp.minimum((H + 15) & ~15, _CAP_ST)
        pos0 = spos_v[pl.ds(0, 16)][0]

        def pad(i, carry):
            for q in range(4):
                stg_v[pl.ds(pl.multiple_of(i * _EMB, _EMB) + q * 16, 16)] = (
                    stg_v[pl.ds(q * 16, 16)])
            plsc.store_scatter(spos_v, [jnp.full((16,), i, i32)],
                               jnp.full((16,), pos0, i32),
                               mask=lanes == 0)
            return carry
        lax.fori_loop(H, Hpad, pad, 0)

        def drain(b, carry):
            o = b * 16
            pv = spos_v[pl.ds(pl.multiple_of(o, 16), 16)] * _EMB
            for j in range(16):
                pltpu.make_async_copy(
                    stg_v.at[pl.ds(pl.multiple_of((o + j) * _EMB, _EMB), _EMB)],
                    uout_hbm.at[pl.ds(pl.multiple_of(pv[j], _EMB), _EMB)],
                    osem).start()
            for j in range(16):
                pltpu.make_async_copy(
                    stg_v.at[pl.ds(0, _EMB)],
                    uout_hbm.at[pl.ds(0, _EMB)], osem).wait()
            return carry
        lax.fori_loop(0, Hpad // 16, drain, 0)

    return gather_user


def _make_sc_game():
    mesh = plsc.VectorSubcoreMesh(core_axis_name="c", subcore_axis_name="s")
    i32 = jnp.int32

    @functools.partial(
        pl.kernel,
        mesh=mesh,
        compiler_params=pltpu.CompilerParams(needs_layout_passes=False),
        out_type=jax.ShapeDtypeStruct((_B, _EMB), jnp.float32),
        scratch_types=[
            pltpu.VMEM((_BPW,), i32),
            pltpu.VMEM((_K, _EMB), jnp.float32),
            pltpu.SemaphoreType.DMA,
        ],
    )
    def gather_game(gidx_hbm, gtab_hbm, gout_hbm, gidx_v, grow_v, gsem):
        wid = lax.axis_index("s") * _NC + lax.axis_index("c")
        gbase = wid * _BPW
        pltpu.sync_copy(gidx_hbm.at[pl.ds(gbase, _BPW)], gidx_v)

        def gchunk(c, carry):
            o = c * _K
            gv = gidx_v[pl.ds(o, _K)]
            for j in range(_K):
                pltpu.make_async_copy(
                    gtab_hbm.at[pl.ds(gv[j], 1)],
                    grow_v.at[pl.ds(j, 1)], gsem).start()
            for j in range(_K):
                pltpu.make_async_copy(
                    gtab_hbm.at[pl.ds(0, 1)],
                    grow_v.at[pl.ds(j, 1)], gsem).wait()
            pltpu.sync_copy(grow_v, gout_hbm.at[pl.ds(gbase + o, _K)])
            return carry
        lax.fori_loop(0, _BPW // _K, gchunk, 0)

    return gather_game


_sc_gather_cache = []


def _sc_gather(uidx, utabT, utail, gidx, gtab):
    if not _sc_gather_cache:
        _sc_gather_cache.append(_make_sc_gather())
        _sc_gather_cache.append(_make_sc_game())
    umf_flat = _sc_gather_cache[0](uidx, utabT, utail)
    gmf = _sc_gather_cache[1](gidx, gtab)
    return umf_flat, gmf


def _dotT(lhsT, rhs):
    return lax.dot_general(lhsT, rhs, (((0,), (0,)), ((), ())),
                           preferred_element_type=jnp.float32)


def _mlp_body(gfT_ref, glT_ref, umf_ref, gmf_ref,
              gw1_ref, gb1_ref, gw2_ref, gb2_ref,
              uw1_ref, ub1_ref, uw2_ref, ub2_ref,
              dw1a_ref, dw1b_ref, db1_ref, dw2_ref, db2_ref,
              out_ref):
    g1 = jnp.maximum(_dotT(gfT_ref[...], gw1_ref[...]) + gb1_ref[...], 0.0)
    genc = jnp.maximum(
        jnp.dot(g1, gw2_ref[...], preferred_element_type=jnp.float32)
        + gb2_ref[...], 0.0)
    u1 = jnp.maximum(_dotT(glT_ref[...], uw1_ref[...]) + ub1_ref[...], 0.0)
    uenc = jnp.maximum(
        jnp.dot(u1, uw2_ref[...], preferred_element_type=jnp.float32)
        + ub2_ref[...], 0.0)
    fu = umf_ref[...] + uenc
    fg = gmf_ref[...] + genc
    h = jnp.maximum(
        jnp.dot(fu, dw1a_ref[...], preferred_element_type=jnp.float32)
        + jnp.dot(fg, dw1b_ref[...], preferred_element_type=jnp.float32)
        + db1_ref[...], 0.0)
    out_ref[...] = (jnp.dot(h, dw2_ref[...], preferred_element_type=jnp.float32)
                    + db2_ref[...])


_R = 2048


def _dense(gfT, glT, umf, gmf, gW1, gb1, gW2, gb2, uW1, ub1, uW2, ub2,
           dW1a, dW1b, db1, dW2, db2):
    nblk = _B // _R

    def cols(i):
        return (0, i)

    def rows(i):
        return (i, 0)

    def whole(i):
        return (0, 0)

    col_spec_feat = pl.BlockSpec((_FEAT, _R), cols)
    row_spec_emb = pl.BlockSpec((_R, _EMB), rows)

    def wspec(a):
        return pl.BlockSpec(a.shape, whole)

    out = pl.pallas_call(
        _mlp_body,
        grid=(nblk,),
        in_specs=[
            col_spec_feat, col_spec_feat, row_spec_emb, row_spec_emb,
            wspec(gW1), wspec(gb1), wspec(gW2), wspec(gb2),
            wspec(uW1), wspec(ub1), wspec(uW2), wspec(ub2),
            wspec(dW1a), wspec(dW1b), wspec(db1), wspec(dW2), wspec(db2),
        ],
        out_specs=pl.BlockSpec((_R, 1), lambda i: (i, 0)),
        out_shape=jax.ShapeDtypeStruct((_B, 1), jnp.float32),
    )(gfT, glT, umf, gmf, gW1, gb1, gW2, gb2, uW1, ub1, uW2, ub2,
      dW1a, dW1b, db1, dW2, db2)
    return out[:, 0]


def kernel(user_input, game_input, game_features, global_features,
           user_table, game_table,
           gW1, gb1, gW2, gb2,
           uW1, ub1, uW2, ub2,
           dW1, db1, dW2, db2):
    utabT = user_table.T
    umf_flat, gmf = _sc_gather(
        user_input, utabT, utabT[:, _NWIN_FULL * _W:], game_input, game_table)
    umf = umf_flat.reshape(_B, _EMB)
    dW1a = dW1[:_EMB]
    dW1b = dW1[_EMB:]
    return _dense(
        game_features.T, global_features.T, umf, gmf,
        gW1, gb1.reshape(1, -1), gW2, gb2.reshape(1, -1),
        uW1, ub1.reshape(1, -1), uW2, ub2.reshape(1, -1),
        dW1a, dW1b, db1.reshape(1, -1), dW2, db2.reshape(1, -1))
